# Initial kernel scaffold; baseline (speedup 1.0000x reference)
import jax
import jax.numpy as jnp
from jax.experimental import pallas as pl


def kernel(x, table):
    raise NotImplementedError("write your pallas kernel here")


# speedup vs baseline: 3.6895x; 3.6895x over previous
import functools
import math

import jax
import jax.numpy as jnp
import numpy as np
from jax import lax
from jax.experimental import pallas as pl
from jax.experimental.pallas import tpu as pltpu
from jax.experimental.pallas import tpu_sc as plsc

_D = 128
_SEQ = 200
_BATCH = 1024
_TOKENS = _BATCH * _SEQ
_SCALE = math.sqrt(float(_D))

_info = plsc.get_sparse_core_info()
_NC, _NS = _info.num_cores, _info.num_subcores
_NW = _NC * _NS
_SEQ_PER_W = _BATCH // _NW
_IDX_MINOR = 100


def _positional(seq, d):
    pe = np.zeros((seq, d), dtype=np.float32)
    position = np.arange(0, seq, dtype=np.float32)[:, None]
    div_term = np.exp(
        np.arange(0, d, 2, dtype=np.float32) * (-math.log(10000.0) / d))
    pe[:, 0::2] = np.sin(position * div_term)
    pe[:, 1::2] = np.cos(position * div_term)
    return pe


_mesh = plsc.VectorSubcoreMesh(core_axis_name="c", subcore_axis_name="s")


@functools.partial(
    pl.kernel,
    out_type=jax.ShapeDtypeStruct((_TOKENS, _D), jnp.float32),
    mesh=_mesh,
    scratch_types=[
        pltpu.VMEM((2, _IDX_MINOR), jnp.int32),
        pltpu.VMEM((_SEQ, _D), jnp.float32),
        pltpu.VMEM((_SEQ, _D), jnp.float32),
        pltpu.SemaphoreType.DMA,
    ],
)
def _emb_kernel(x_hbm, table_hbm, pe_hbm, out_hbm, idx_v, rows_v, pe_v, sem):
    wid = lax.axis_index("s") * _NC + lax.axis_index("c")
    pltpu.sync_copy(pe_hbm, pe_v)

    def seq_body(i, carry):
        seq_id = wid * _SEQ_PER_W + i
        pltpu.sync_copy(x_hbm.at[pl.ds(seq_id * 2, 2)], idx_v)
        g0 = pltpu.async_copy(
            table_hbm.at[idx_v.at[0]], rows_v.at[pl.ds(0, _IDX_MINOR)], sem)
        g1 = pltpu.async_copy(
            table_hbm.at[idx_v.at[1]],
            rows_v.at[pl.ds(_IDX_MINOR, _IDX_MINOR)], sem)
        g0.wait()
        g1.wait()

        def row_body(s, c2):
            for j in range(_D // 16):
                sl = pl.ds(j * 16, 16)
                rows_v[s, sl] = rows_v[s, sl] * _SCALE + pe_v[s, sl]
            return c2

        lax.fori_loop(0, _SEQ, row_body, 0)
        pltpu.sync_copy(rows_v, out_hbm.at[pl.ds(seq_id * _SEQ, _SEQ)])
        return carry

    lax.fori_loop(0, _SEQ_PER_W, seq_body, 0)


def kernel(x, table):
    x_flat = x.astype(jnp.int32).reshape(_TOKENS // _IDX_MINOR, _IDX_MINOR)
    pe = jnp.asarray(_positional(_SEQ, _D))
    out = _emb_kernel(x_flat, table, pe)
    return out.reshape(_BATCH, _SEQ, _D)


# speedup vs baseline: 4.5142x; 1.2235x over previous
import functools
import math

import jax
import jax.numpy as jnp
import numpy as np
from jax import lax
from jax.experimental import pallas as pl
from jax.experimental.pallas import tpu as pltpu
from jax.experimental.pallas import tpu_sc as plsc

_D = 128
_SEQ = 200
_BATCH = 1024
_TOKENS = _BATCH * _SEQ
_SCALE = math.sqrt(float(_D))

_info = plsc.get_sparse_core_info()
_NC, _NS = _info.num_cores, _info.num_subcores
_NW = _NC * _NS
_SEQ_PER_W = _BATCH // _NW
_IDX_MINOR = 100
_NBUF = 3


def _positional(seq, d):
    pe = np.zeros((seq, d), dtype=np.float32)
    position = np.arange(0, seq, dtype=np.float32)[:, None]
    div_term = np.exp(
        np.arange(0, d, 2, dtype=np.float32) * (-math.log(10000.0) / d))
    pe[:, 0::2] = np.sin(position * div_term)
    pe[:, 1::2] = np.cos(position * div_term)
    return pe


_mesh = plsc.VectorSubcoreMesh(core_axis_name="c", subcore_axis_name="s")


@functools.partial(
    pl.kernel,
    out_type=jax.ShapeDtypeStruct((_TOKENS, _D), jnp.float32),
    mesh=_mesh,
    scratch_types=[
        pltpu.VMEM((2 * _SEQ_PER_W, _IDX_MINOR), jnp.int32),
        pltpu.VMEM((_SEQ, _D), jnp.float32),
        pltpu.VMEM((_SEQ, _D), jnp.float32),
        pltpu.VMEM((_SEQ, _D), jnp.float32),
        pltpu.VMEM((_SEQ, _D), jnp.float32),
        pltpu.SemaphoreType.DMA,
        pltpu.SemaphoreType.DMA,
        pltpu.SemaphoreType.DMA,
        pltpu.SemaphoreType.DMA,
        pltpu.SemaphoreType.DMA,
        pltpu.SemaphoreType.DMA,
    ],
)
def _emb_kernel(x_hbm, table_hbm, pe_hbm, out_hbm,
                idx_v, b0, b1, b2, pe_v, g0, g1, g2, s0, s1, s2):
    wid = lax.axis_index("s") * _NC + lax.axis_index("c")
    seq0 = wid * _SEQ_PER_W
    pltpu.sync_copy(x_hbm.at[pl.ds(seq0 * 2, 2 * _SEQ_PER_W)], idx_v)
    pltpu.sync_copy(pe_hbm, pe_v)

    bufs = (b0, b1, b2)
    gsems = (g0, g1, g2)
    ssems = (s0, s1, s2)

    def fire_gather(h):
        p = h % _NBUF
        c0 = pltpu.async_copy(
            table_hbm.at[idx_v.at[2 * h]],
            bufs[p].at[pl.ds(0, _IDX_MINOR)], gsems[p])
        c1 = pltpu.async_copy(
            table_hbm.at[idx_v.at[2 * h + 1]],
            bufs[p].at[pl.ds(_IDX_MINOR, _IDX_MINOR)], gsems[p])
        return (c0, c1)

    def fire_scatter(h):
        p = h % _NBUF
        return pltpu.async_copy(
            bufs[p], out_hbm.at[pl.ds((seq0 + h) * _SEQ, _SEQ)], ssems[p])

    def compute(h):
        buf = bufs[h % _NBUF]

        def row_body(s, c2):
            for j in range(_D // 16):
                sl = pl.ds(j * 16, 16)
                buf[s, sl] = buf[s, sl] * _SCALE + pe_v[s, sl]
            return c2

        lax.fori_loop(0, _SEQ, row_body, 0)

    gathers = {0: fire_gather(0), 1: fire_gather(1)}
    scatters = {}
    for h in range(_SEQ_PER_W):
        for c in gathers.pop(h):
            c.wait()
        compute(h)
        scatters[h] = fire_scatter(h)
        if h >= 1:
            scatters.pop(h - 1).wait()
        if h + 2 < _SEQ_PER_W:
            gathers[h + 2] = fire_gather(h + 2)
    scatters.pop(_SEQ_PER_W - 1).wait()


def kernel(x, table):
    x_flat = x.astype(jnp.int32).reshape(_TOKENS // _IDX_MINOR, _IDX_MINOR)
    pe = jnp.asarray(_positional(_SEQ, _D))
    out = _emb_kernel(x_flat, table, pe)
    return out.reshape(_BATCH, _SEQ, _D)
